# baseline (speedup 1.0000x reference)
import functools

import jax
import jax.numpy as jnp
from jax import lax
from jax.experimental import pallas as pl
from jax.experimental.pallas import tpu as pltpu
from jax.experimental.pallas import tpu_sc as plsc

N = 10000
E = 320000
D = 128
DH = D // 2

_info = plsc.get_sparse_core_info()
NC = _info.num_cores
NS = _info.num_subcores
NW = NC * NS
EPT = E // NW
CH = 48
NCH = 210
EPT_P = NCH * CH
NP = 10240
RPS = NP // NS
ZR = 160

_mesh = plsc.VectorSubcoreMesh(core_axis_name="c", subcore_axis_name="s")
_sc_params = pltpu.CompilerParams(needs_layout_passes=False,
                                  use_tc_tiling_on_sc=False)


@functools.partial(
    pl.kernel,
    mesh=_mesh,
    out_type=jax.ShapeDtypeStruct((2, NW, RPS, DH), jnp.float32),
    compiler_params=_sc_params,
    scratch_types=[
        pltpu.VMEM((2, NCH, CH), jnp.int32),
        pltpu.VMEM((NCH, CH), jnp.int32),
        pltpu.VMEM((CH, DH), jnp.float32),
        pltpu.VMEM((CH, DH), jnp.float32),
        pltpu.VMEM((ZR, DH), jnp.float32),
        pltpu.VMEM_SHARED((NP, DH), jnp.float32),
        pltpu.SemaphoreType.DMA,
        pltpu.SemaphoreType.DMA,
        pltpu.SemaphoreType.DMA,
        pltpu.SemaphoreType.DMA,
        pltpu.SemaphoreType.DMA,
    ],
)
def _seg_sum(z_hbm, src2_hbm, dst_hbm, out_hbm,
             srcv, dstv, rows0, rows1, zbuf, acc,
             sem_g0, sem_g1, sem_s0, sem_s1, sem_z):
    c = lax.axis_index("c")
    s = lax.axis_index("s")
    wid = c * NS + s
    base_row = s * RPS

    def zfill(i, carry):
        for j in range(DH // 16):
            zbuf[i, pl.ds(j * 16, 16)] = jnp.zeros((16,), jnp.float32)
        return carry
    lax.fori_loop(0, ZR, zfill, 0)

    pltpu.sync_copy(src2_hbm.at[wid], srcv)
    pltpu.sync_copy(dst_hbm.at[wid], dstv)

    for h in range(2):
        for i in range(RPS // ZR):
            pltpu.async_copy(zbuf, acc.at[pl.ds(base_row + i * ZR, ZR)],
                             sem_z)
        for i in range(RPS // ZR):
            pltpu.make_async_copy(
                zbuf, acc.at[pl.ds(base_row + i * ZR, ZR)], sem_z).wait()
        plsc.subcore_barrier()

        pltpu.async_copy(z_hbm.at[srcv.at[h, 0]], rows0, sem_g0)
        pltpu.async_copy(z_hbm.at[srcv.at[h, 1]], rows1, sem_g1)

        def body(i, carry):
            j = 2 * i
            pltpu.make_async_copy(
                z_hbm.at[srcv.at[h, j]], rows0, sem_g0).wait()
            pltpu.async_copy(rows0, acc.at[dstv.at[j]], sem_s0, add=True)
            pltpu.make_async_copy(
                z_hbm.at[srcv.at[h, j + 1]], rows1, sem_g1).wait()
            pltpu.async_copy(rows1, acc.at[dstv.at[j + 1]], sem_s1, add=True)

            pltpu.make_async_copy(rows0, acc.at[dstv.at[j]], sem_s0).wait()

            @pl.when(j + 2 < NCH)
            def _():
                pltpu.async_copy(z_hbm.at[srcv.at[h, j + 2]], rows0, sem_g0)

            pltpu.make_async_copy(
                rows1, acc.at[dstv.at[j + 1]], sem_s1).wait()

            @pl.when(j + 3 < NCH)
            def _():
                pltpu.async_copy(z_hbm.at[srcv.at[h, j + 3]], rows1, sem_g1)
            return carry

        lax.fori_loop(0, NCH // 2, body, 0)
        plsc.subcore_barrier()

        pltpu.sync_copy(acc.at[pl.ds(base_row, RPS)], out_hbm.at[h, wid])


@functools.partial(
    pl.kernel,
    mesh=_mesh,
    out_type=jax.ShapeDtypeStruct((3, NW, N), jnp.float32),
    compiler_params=_sc_params,
    scratch_types=[
        pltpu.VMEM((EPT,), jnp.int32),
        pltpu.VMEM((N,), jnp.float32),
    ],
)
def _degree_counts(dst_hbm, out_hbm, dstf, cntv):
    c = lax.axis_index("c")
    s = lax.axis_index("s")
    wid = c * NS + s
    for r in range(3):
        def zero(i, carry):
            cntv[pl.ds(i * 16, 16)] = jnp.zeros((16,), jnp.float32)
            return carry
        lax.fori_loop(0, N // 16, zero, 0)
        pltpu.sync_copy(dst_hbm.at[r, wid], dstf)

        def body(i, carry):
            d = dstf[pl.ds(i * 16, 16)]
            plsc.addupdate_scatter(cntv, [d], jnp.ones((16,), jnp.float32))
            return carry
        lax.fori_loop(0, EPT // 16, body, 0)
        pltpu.sync_copy(cntv, out_hbm.at[r, wid])


TB = 2000


def _lin3_body(x0, w0, x1, w1, x2, w2, o0, o1, o2):
    o0[...] = jnp.dot(x0[...], w0[...], preferred_element_type=jnp.float32)
    o1[...] = jnp.dot(x1[...], w1[...], preferred_element_type=jnp.float32)
    o2[...] = jnp.dot(x2[...], w2[...], preferred_element_type=jnp.float32)


def _agg_of(p0, p1, inv):
    return jnp.concatenate([p0[0] + p0[1], p1[0] + p1[1]], axis=-1) * inv[...]


def _combine_body(p0, p1, inv, x, wr, b, w2l, h_o, z2_o):
    agg = _agg_of(p0, p1, inv)
    h = jnp.maximum(
        agg + b[...] + jnp.dot(x[...], wr[...],
                               preferred_element_type=jnp.float32), 0.0)
    h_o[...] = h
    z2_o[...] = jnp.dot(h, w2l[...], preferred_element_type=jnp.float32)


def _final_body(pu0, pu1, iu, hu, wu, bu, ps0, ps1, is_, hs, ws, bs,
                pk0, pk1, ik, hk, wk, bk, out):
    def term(p0, p1, inv, h, w, b):
        return _agg_of(p0, p1, inv) + b[...] + jnp.dot(
            h[...], w[...], preferred_element_type=jnp.float32)
    out[...] = term(pu0, pu1, iu, hu, wu, bu) + 0.5 * (
        term(ps0, ps1, is_, hs, ws, bs) + term(pk0, pk1, ik, hk, wk, bk))


_xspec = pl.BlockSpec((TB, D), lambda i: (i, 0))
_pspec = pl.BlockSpec((NC, TB, DH), lambda i: (0, i, 0))
_ispec = pl.BlockSpec((TB, 1), lambda i: (i, 0))
_wspec = pl.BlockSpec((D, D), lambda i: (0, 0))
_bspec = pl.BlockSpec((1, D), lambda i: (0, 0))
_GRID = (N // TB,)
_osd = jax.ShapeDtypeStruct((N, D), jnp.float32)

_lin3 = pl.pallas_call(
    _lin3_body,
    grid=_GRID,
    in_specs=[_xspec, _wspec] * 3,
    out_specs=[_xspec] * 3,
    out_shape=[_osd] * 3,
)

_combine = pl.pallas_call(
    _combine_body,
    grid=_GRID,
    in_specs=[_pspec, _pspec, _ispec, _xspec, _wspec, _bspec, _wspec],
    out_specs=[_xspec, _xspec],
    out_shape=[_osd, _osd],
)

_final = pl.pallas_call(
    _final_body,
    grid=_GRID,
    in_specs=[_pspec, _pspec, _ispec, _xspec, _wspec, _bspec] * 3,
    out_specs=_xspec,
    out_shape=_osd,
)


def kernel(ui_x, s_x, k_x, ui_edge_index, s_edge_index, k_edge_index,
           ui_W1l, ui_b1l, ui_W1r, ui_W2l, ui_b2l, ui_W2r,
           s_W1l, s_b1l, s_W1r, s_W2l, s_b2l, s_W2r,
           k_W1l, k_b1l, k_W1r, k_W2l, k_b2l, k_W2r):
    xs = (ui_x, s_x, k_x)
    eis = (ui_edge_index, s_edge_index, k_edge_index)
    W1l = (ui_W1l, s_W1l, k_W1l)
    b1l = (ui_b1l, s_b1l, k_b1l)
    W1r = (ui_W1r, s_W1r, k_W1r)
    W2l = (ui_W2l, s_W2l, k_W2l)
    b2l = (ui_b2l, s_b2l, k_b2l)
    W2r = (ui_W2r, s_W2r, k_W2r)

    PADW = EPT_P - EPT
    if PADW:
        pad_dst = (N + (jnp.arange(PADW)[None, :]
                        + 7 * jnp.arange(NW)[:, None]) % (NP - N)
                   ).astype(jnp.int32)
    src2s, dsts = [], []
    for ei in eis:
        src = ei[0].astype(jnp.int32).reshape(NW, EPT)
        src = jnp.pad(src, ((0, 0), (0, PADW)))
        src2s.append(jnp.stack([2 * src, 2 * src + 1],
                               axis=1).reshape(NW, 2, NCH, CH))
        dst = ei[1].astype(jnp.int32).reshape(NW, EPT)
        if PADW:
            dst = jnp.concatenate([dst, pad_dst], axis=1)
        dsts.append(dst.reshape(NW, NCH, CH))
    dst_flat = jnp.stack([ei[1].astype(jnp.int32).reshape(NW, EPT)
                          for ei in eis])

    cnt_part = _degree_counts(dst_flat)
    cnt = cnt_part.sum(axis=1)
    inv = 1.0 / jnp.clip(cnt, 1.0, None)
    invs = [inv[r][:, None] for r in range(3)]

    z1 = _lin3(xs[0], W1l[0], xs[1], W1l[1], xs[2], W1l[2])

    def chained_seg(z, r, tok):
        z, _ = lax.optimization_barrier((z, tok))
        p = _seg_sum(z.reshape(2 * N, DH), src2s[r], dsts[r])
        halves = (p[0].reshape(NC, NP, DH), p[1].reshape(NC, NP, DH))
        return halves, p[0, 0, 0, :8]

    tok = cnt_part[0, 0, :8]
    p1s, hs, z2s, p2s = [], [], [], []
    for r in range(3):
        p1, tok = chained_seg(z1[r], r, tok)
        p1s.append(p1)
    for r in range(3):
        h, z2 = _combine(p1s[r][0], p1s[r][1], invs[r], xs[r], W1r[r],
                         b1l[r].reshape(1, D), W2l[r])
        hs.append(h)
        z2s.append(z2)
    for r in range(3):
        p2, tok = chained_seg(z2s[r], r, tok)
        p2s.append(p2)

    return _final(
        p2s[0][0], p2s[0][1], invs[0], hs[0], W2r[0], b2l[0].reshape(1, D),
        p2s[1][0], p2s[1][1], invs[1], hs[1], W2r[1], b2l[1].reshape(1, D),
        p2s[2][0], p2s[2][1], invs[2], hs[2], W2r[2], b2l[2].reshape(1, D),
    )


# speedup vs baseline: 1.8409x; 1.8409x over previous
import functools

import jax
import jax.numpy as jnp
from jax import lax
from jax.experimental import pallas as pl
from jax.experimental.pallas import tpu as pltpu
from jax.experimental.pallas import tpu_sc as plsc

N = 10000
E = 320000
D = 128
DH = D // 2

_info = plsc.get_sparse_core_info()
NC = _info.num_cores
NS = _info.num_subcores
NW = NC * NS
EPT = E // NW
CH = 40
NCH = 250
NCH4 = (NCH // 4) * 4
EPT_P = NCH * CH
NP = 10240
RPS = NP // NS
ZR = 160

_mesh = plsc.VectorSubcoreMesh(core_axis_name="c", subcore_axis_name="s")
_sc_params = pltpu.CompilerParams(needs_layout_passes=False,
                                  use_tc_tiling_on_sc=False)


@functools.partial(
    pl.kernel,
    mesh=_mesh,
    out_type=jax.ShapeDtypeStruct((2, NW, RPS, DH), jnp.float32),
    compiler_params=_sc_params,
    scratch_types=[
        pltpu.VMEM((2, NCH, CH), jnp.int32),
        pltpu.VMEM((NCH, CH), jnp.int32),
        pltpu.VMEM((CH, DH), jnp.float32),
        pltpu.VMEM((CH, DH), jnp.float32),
        pltpu.VMEM((CH, DH), jnp.float32),
        pltpu.VMEM((CH, DH), jnp.float32),
        pltpu.VMEM((ZR, DH), jnp.float32),
        pltpu.VMEM_SHARED((NP, DH), jnp.float32),
        pltpu.SemaphoreType.DMA,
        pltpu.SemaphoreType.DMA,
        pltpu.SemaphoreType.DMA,
        pltpu.SemaphoreType.DMA,
        pltpu.SemaphoreType.DMA,
        pltpu.SemaphoreType.DMA,
        pltpu.SemaphoreType.DMA,
        pltpu.SemaphoreType.DMA,
        pltpu.SemaphoreType.DMA,
    ],
)
def _seg_sum(z_hbm, src2_hbm, dst_hbm, out_hbm,
             srcv, dstv, rows0, rows1, rows2, rows3, zbuf, acc,
             sem_g0, sem_g1, sem_g2, sem_g3,
             sem_s0, sem_s1, sem_s2, sem_s3, sem_z):
    c = lax.axis_index("c")
    s = lax.axis_index("s")
    wid = c * NS + s
    base_row = s * RPS

    def zfill(i, carry):
        for j in range(DH // 16):
            zbuf[i, pl.ds(j * 16, 16)] = jnp.zeros((16,), jnp.float32)
        return carry
    lax.fori_loop(0, ZR, zfill, 0)

    pltpu.sync_copy(src2_hbm.at[wid], srcv)
    pltpu.sync_copy(dst_hbm.at[wid], dstv)

    for h in range(2):
        for i in range(RPS // ZR):
            pltpu.async_copy(zbuf, acc.at[pl.ds(base_row + i * ZR, ZR)],
                             sem_z)
        for i in range(RPS // ZR):
            pltpu.make_async_copy(
                zbuf, acc.at[pl.ds(base_row + i * ZR, ZR)], sem_z).wait()
        plsc.subcore_barrier()

        bufs = (rows0, rows1, rows2, rows3)
        gsems = (sem_g0, sem_g1, sem_g2, sem_g3)
        ssems = (sem_s0, sem_s1, sem_s2, sem_s3)
        for k in range(4):
            pltpu.async_copy(z_hbm.at[srcv.at[h, k]], bufs[k], gsems[k])

        def body(i, carry):
            j = 4 * i
            for k in range(4):
                pltpu.make_async_copy(
                    z_hbm.at[srcv.at[h, j + k]], bufs[k], gsems[k]).wait()
                pltpu.async_copy(bufs[k], acc.at[dstv.at[j + k]],
                                 ssems[k], add=True)
            for k in range(4):
                pltpu.make_async_copy(
                    bufs[k], acc.at[dstv.at[j + k]], ssems[k]).wait()

                @pl.when(j + 4 + k < NCH)
                def _():
                    pltpu.async_copy(z_hbm.at[srcv.at[h, j + 4 + k]],
                                     bufs[k], gsems[k])
            return carry

        lax.fori_loop(0, NCH4 // 4, body, 0)

        for k in range(NCH - NCH4):
            pltpu.make_async_copy(
                z_hbm.at[srcv.at[h, NCH4 + k]], bufs[k], gsems[k]).wait()
            pltpu.async_copy(bufs[k], acc.at[dstv.at[NCH4 + k]],
                             ssems[k], add=True)
        for k in range(NCH - NCH4):
            pltpu.make_async_copy(
                bufs[k], acc.at[dstv.at[NCH4 + k]], ssems[k]).wait()
        plsc.subcore_barrier()

        pltpu.sync_copy(acc.at[pl.ds(base_row, RPS)], out_hbm.at[h, wid])


@functools.partial(
    pl.kernel,
    mesh=_mesh,
    out_type=jax.ShapeDtypeStruct((3, NW, N), jnp.float32),
    compiler_params=_sc_params,
    scratch_types=[
        pltpu.VMEM((EPT,), jnp.int32),
        pltpu.VMEM((N,), jnp.float32),
    ],
)
def _degree_counts(dst_hbm, out_hbm, dstf, cntv):
    c = lax.axis_index("c")
    s = lax.axis_index("s")
    wid = c * NS + s
    for r in range(3):
        def zero(i, carry):
            cntv[pl.ds(i * 16, 16)] = jnp.zeros((16,), jnp.float32)
            return carry
        lax.fori_loop(0, N // 16, zero, 0)
        pltpu.sync_copy(dst_hbm.at[r, wid], dstf)

        def body(i, carry):
            d = dstf[pl.ds(i * 16, 16)]
            plsc.addupdate_scatter(cntv, [d], jnp.ones((16,), jnp.float32))
            return carry
        lax.fori_loop(0, EPT // 16, body, 0)
        pltpu.sync_copy(cntv, out_hbm.at[r, wid])


TB = 2000


def _lin3_body(x0, w0, x1, w1, x2, w2, o0, o1, o2):
    o0[...] = jnp.dot(x0[...], w0[...], preferred_element_type=jnp.float32)
    o1[...] = jnp.dot(x1[...], w1[...], preferred_element_type=jnp.float32)
    o2[...] = jnp.dot(x2[...], w2[...], preferred_element_type=jnp.float32)


def _agg_of(p0, p1, inv):
    return jnp.concatenate([p0[0] + p0[1], p1[0] + p1[1]], axis=-1) * inv[...]


def _combine_body(p0, p1, inv, x, wr, b, w2l, h_o, z2_o):
    agg = _agg_of(p0, p1, inv)
    h = jnp.maximum(
        agg + b[...] + jnp.dot(x[...], wr[...],
                               preferred_element_type=jnp.float32), 0.0)
    h_o[...] = h
    z2_o[...] = jnp.dot(h, w2l[...], preferred_element_type=jnp.float32)


def _final_body(pu0, pu1, iu, hu, wu, bu, ps0, ps1, is_, hs, ws, bs,
                pk0, pk1, ik, hk, wk, bk, out):
    def term(p0, p1, inv, h, w, b):
        return _agg_of(p0, p1, inv) + b[...] + jnp.dot(
            h[...], w[...], preferred_element_type=jnp.float32)
    out[...] = term(pu0, pu1, iu, hu, wu, bu) + 0.5 * (
        term(ps0, ps1, is_, hs, ws, bs) + term(pk0, pk1, ik, hk, wk, bk))


_xspec = pl.BlockSpec((TB, D), lambda i: (i, 0))
_pspec = pl.BlockSpec((NC, TB, DH), lambda i: (0, i, 0))
_ispec = pl.BlockSpec((TB, 1), lambda i: (i, 0))
_wspec = pl.BlockSpec((D, D), lambda i: (0, 0))
_bspec = pl.BlockSpec((1, D), lambda i: (0, 0))
_GRID = (N // TB,)
_osd = jax.ShapeDtypeStruct((N, D), jnp.float32)

_lin3 = pl.pallas_call(
    _lin3_body,
    grid=_GRID,
    in_specs=[_xspec, _wspec] * 3,
    out_specs=[_xspec] * 3,
    out_shape=[_osd] * 3,
)

_combine = pl.pallas_call(
    _combine_body,
    grid=_GRID,
    in_specs=[_pspec, _pspec, _ispec, _xspec, _wspec, _bspec, _wspec],
    out_specs=[_xspec, _xspec],
    out_shape=[_osd, _osd],
)

_final = pl.pallas_call(
    _final_body,
    grid=_GRID,
    in_specs=[_pspec, _pspec, _ispec, _xspec, _wspec, _bspec] * 3,
    out_specs=_xspec,
    out_shape=_osd,
)


def kernel(ui_x, s_x, k_x, ui_edge_index, s_edge_index, k_edge_index,
           ui_W1l, ui_b1l, ui_W1r, ui_W2l, ui_b2l, ui_W2r,
           s_W1l, s_b1l, s_W1r, s_W2l, s_b2l, s_W2r,
           k_W1l, k_b1l, k_W1r, k_W2l, k_b2l, k_W2r):
    xs = (ui_x, s_x, k_x)
    eis = (ui_edge_index, s_edge_index, k_edge_index)
    W1l = (ui_W1l, s_W1l, k_W1l)
    b1l = (ui_b1l, s_b1l, k_b1l)
    W1r = (ui_W1r, s_W1r, k_W1r)
    W2l = (ui_W2l, s_W2l, k_W2l)
    b2l = (ui_b2l, s_b2l, k_b2l)
    W2r = (ui_W2r, s_W2r, k_W2r)

    PADW = EPT_P - EPT
    if PADW:
        pad_dst = (N + (jnp.arange(PADW)[None, :]
                        + 7 * jnp.arange(NW)[:, None]) % (NP - N)
                   ).astype(jnp.int32)
    src2s, dsts = [], []
    for ei in eis:
        src = ei[0].astype(jnp.int32).reshape(NW, EPT)
        src = jnp.pad(src, ((0, 0), (0, PADW)))
        src2s.append(jnp.stack([2 * src, 2 * src + 1],
                               axis=1).reshape(NW, 2, NCH, CH))
        dst = ei[1].astype(jnp.int32).reshape(NW, EPT)
        if PADW:
            dst = jnp.concatenate([dst, pad_dst], axis=1)
        dsts.append(dst.reshape(NW, NCH, CH))
    dst_flat = jnp.stack([ei[1].astype(jnp.int32).reshape(NW, EPT)
                          for ei in eis])

    cnt_part = _degree_counts(dst_flat)
    cnt = cnt_part.sum(axis=1)
    inv = 1.0 / jnp.clip(cnt, 1.0, None)
    invs = [inv[r][:, None] for r in range(3)]

    z1 = _lin3(xs[0], W1l[0], xs[1], W1l[1], xs[2], W1l[2])

    def chained_seg(z, r, tok):
        z, _ = lax.optimization_barrier((z, tok))
        p = _seg_sum(z.reshape(2 * N, DH), src2s[r], dsts[r])
        halves = (p[0].reshape(NC, NP, DH), p[1].reshape(NC, NP, DH))
        return halves, p[0, 0, 0, :8]

    tok = cnt_part[0, 0, :8]
    p1s, hs, z2s, p2s = [], [], [], []
    for r in range(3):
        p1, tok = chained_seg(z1[r], r, tok)
        p1s.append(p1)
    for r in range(3):
        h, z2 = _combine(p1s[r][0], p1s[r][1], invs[r], xs[r], W1r[r],
                         b1l[r].reshape(1, D), W2l[r])
        hs.append(h)
        z2s.append(z2)
    for r in range(3):
        p2, tok = chained_seg(z2s[r], r, tok)
        p2s.append(p2)

    return _final(
        p2s[0][0], p2s[0][1], invs[0], hs[0], W2r[0], b2l[0].reshape(1, D),
        p2s[1][0], p2s[1][1], invs[1], hs[1], W2r[1], b2l[1].reshape(1, D),
        p2s[2][0], p2s[2][1], invs[2], hs[2], W2r[2], b2l[2].reshape(1, D),
    )


# speedup vs baseline: 2.2424x; 1.2181x over previous
import functools

import jax
import jax.numpy as jnp
from jax import lax
from jax.experimental import pallas as pl
from jax.experimental.pallas import tpu as pltpu
from jax.experimental.pallas import tpu_sc as plsc

N = 10000
E = 320000
D = 128
DH = D // 2

_info = plsc.get_sparse_core_info()
NC = _info.num_cores
NS = _info.num_subcores
NW = NC * NS
EPT = E // NW
CH = 40
NCH = 250
DEPTH = 8
NCHD = (NCH // DEPTH) * DEPTH
EPT_P = NCH * CH
NP = 10240
RPS = NP // NS
ZR = 160

_mesh = plsc.VectorSubcoreMesh(core_axis_name="c", subcore_axis_name="s")
_sc_params = pltpu.CompilerParams(needs_layout_passes=False,
                                  use_tc_tiling_on_sc=False)


@functools.partial(
    pl.kernel,
    mesh=_mesh,
    out_type=jax.ShapeDtypeStruct((2, NW, RPS, DH), jnp.float32),
    compiler_params=_sc_params,
    scratch_types=[
        pltpu.VMEM((2, NCH, CH), jnp.int32),
        pltpu.VMEM((NCH, CH), jnp.int32),
        *[pltpu.VMEM((CH, DH), jnp.float32) for _ in range(DEPTH)],
        pltpu.VMEM((ZR, DH), jnp.float32),
        pltpu.VMEM_SHARED((NP, DH), jnp.float32),
        *[pltpu.SemaphoreType.DMA for _ in range(DEPTH)],
        *[pltpu.SemaphoreType.DMA for _ in range(DEPTH)],
        pltpu.SemaphoreType.DMA,
    ],
)
def _seg_sum(z_hbm, src2_hbm, dst_hbm, out_hbm, srcv, dstv, *rest):
    bufs = rest[:DEPTH]
    zbuf = rest[DEPTH]
    acc = rest[DEPTH + 1]
    gsems = rest[DEPTH + 2:2 * DEPTH + 2]
    ssems = rest[2 * DEPTH + 2:3 * DEPTH + 2]
    sem_z = rest[3 * DEPTH + 2]
    c = lax.axis_index("c")
    s = lax.axis_index("s")
    wid = c * NS + s
    base_row = s * RPS

    def zfill(i, carry):
        for j in range(DH // 16):
            zbuf[i, pl.ds(j * 16, 16)] = jnp.zeros((16,), jnp.float32)
        return carry
    lax.fori_loop(0, ZR, zfill, 0)

    pltpu.sync_copy(src2_hbm.at[wid], srcv)
    pltpu.sync_copy(dst_hbm.at[wid], dstv)

    for h in range(2):
        for i in range(RPS // ZR):
            pltpu.async_copy(zbuf, acc.at[pl.ds(base_row + i * ZR, ZR)],
                             sem_z)
        for i in range(RPS // ZR):
            pltpu.make_async_copy(
                zbuf, acc.at[pl.ds(base_row + i * ZR, ZR)], sem_z).wait()
        plsc.subcore_barrier()

        for k in range(DEPTH):
            pltpu.async_copy(z_hbm.at[srcv.at[h, k]], bufs[k], gsems[k])

        def body(i, carry):
            j = DEPTH * i
            for k in range(DEPTH):
                pltpu.make_async_copy(
                    z_hbm.at[srcv.at[h, j + k]], bufs[k], gsems[k]).wait()
                pltpu.async_copy(bufs[k], acc.at[dstv.at[j + k]],
                                 ssems[k], add=True)
            for k in range(DEPTH):
                pltpu.make_async_copy(
                    bufs[k], acc.at[dstv.at[j + k]], ssems[k]).wait()

                @pl.when(j + DEPTH + k < NCH)
                def _():
                    pltpu.async_copy(z_hbm.at[srcv.at[h, j + DEPTH + k]],
                                     bufs[k], gsems[k])
            return carry

        lax.fori_loop(0, NCHD // DEPTH, body, 0)

        for k in range(NCH - NCHD):
            pltpu.make_async_copy(
                z_hbm.at[srcv.at[h, NCHD + k]], bufs[k], gsems[k]).wait()
            pltpu.async_copy(bufs[k], acc.at[dstv.at[NCHD + k]],
                             ssems[k], add=True)
        for k in range(NCH - NCHD):
            pltpu.make_async_copy(
                bufs[k], acc.at[dstv.at[NCHD + k]], ssems[k]).wait()
        plsc.subcore_barrier()

        pltpu.sync_copy(acc.at[pl.ds(base_row, RPS)], out_hbm.at[h, wid])


@functools.partial(
    pl.kernel,
    mesh=_mesh,
    out_type=jax.ShapeDtypeStruct((3, NW, N), jnp.float32),
    compiler_params=_sc_params,
    scratch_types=[
        pltpu.VMEM((EPT,), jnp.int32),
        pltpu.VMEM((N,), jnp.float32),
    ],
)
def _degree_counts(dst_hbm, out_hbm, dstf, cntv):
    c = lax.axis_index("c")
    s = lax.axis_index("s")
    wid = c * NS + s
    for r in range(3):
        def zero(i, carry):
            cntv[pl.ds(i * 16, 16)] = jnp.zeros((16,), jnp.float32)
            return carry
        lax.fori_loop(0, N // 16, zero, 0)
        pltpu.sync_copy(dst_hbm.at[r, wid], dstf)

        def body(i, carry):
            d = dstf[pl.ds(i * 16, 16)]
            plsc.addupdate_scatter(cntv, [d], jnp.ones((16,), jnp.float32))
            return carry
        lax.fori_loop(0, EPT // 16, body, 0)
        pltpu.sync_copy(cntv, out_hbm.at[r, wid])


TB = 2000


def _lin3_body(x0, w0, x1, w1, x2, w2, o0, o1, o2):
    o0[...] = jnp.dot(x0[...], w0[...], preferred_element_type=jnp.float32)
    o1[...] = jnp.dot(x1[...], w1[...], preferred_element_type=jnp.float32)
    o2[...] = jnp.dot(x2[...], w2[...], preferred_element_type=jnp.float32)


def _agg_of(p0, p1, inv):
    return jnp.concatenate([p0[0] + p0[1], p1[0] + p1[1]], axis=-1) * inv[...]


def _combine_body(p0, p1, inv, x, wr, b, w2l, h_o, z2_o):
    agg = _agg_of(p0, p1, inv)
    h = jnp.maximum(
        agg + b[...] + jnp.dot(x[...], wr[...],
                               preferred_element_type=jnp.float32), 0.0)
    h_o[...] = h
    z2_o[...] = jnp.dot(h, w2l[...], preferred_element_type=jnp.float32)


def _final_body(pu0, pu1, iu, hu, wu, bu, ps0, ps1, is_, hs, ws, bs,
                pk0, pk1, ik, hk, wk, bk, out):
    def term(p0, p1, inv, h, w, b):
        return _agg_of(p0, p1, inv) + b[...] + jnp.dot(
            h[...], w[...], preferred_element_type=jnp.float32)
    out[...] = term(pu0, pu1, iu, hu, wu, bu) + 0.5 * (
        term(ps0, ps1, is_, hs, ws, bs) + term(pk0, pk1, ik, hk, wk, bk))


_xspec = pl.BlockSpec((TB, D), lambda i: (i, 0))
_pspec = pl.BlockSpec((NC, TB, DH), lambda i: (0, i, 0))
_ispec = pl.BlockSpec((TB, 1), lambda i: (i, 0))
_wspec = pl.BlockSpec((D, D), lambda i: (0, 0))
_bspec = pl.BlockSpec((1, D), lambda i: (0, 0))
_GRID = (N // TB,)
_osd = jax.ShapeDtypeStruct((N, D), jnp.float32)

_lin3 = pl.pallas_call(
    _lin3_body,
    grid=_GRID,
    in_specs=[_xspec, _wspec] * 3,
    out_specs=[_xspec] * 3,
    out_shape=[_osd] * 3,
)

_combine = pl.pallas_call(
    _combine_body,
    grid=_GRID,
    in_specs=[_pspec, _pspec, _ispec, _xspec, _wspec, _bspec, _wspec],
    out_specs=[_xspec, _xspec],
    out_shape=[_osd, _osd],
)

_final = pl.pallas_call(
    _final_body,
    grid=_GRID,
    in_specs=[_pspec, _pspec, _ispec, _xspec, _wspec, _bspec] * 3,
    out_specs=_xspec,
    out_shape=_osd,
)


def kernel(ui_x, s_x, k_x, ui_edge_index, s_edge_index, k_edge_index,
           ui_W1l, ui_b1l, ui_W1r, ui_W2l, ui_b2l, ui_W2r,
           s_W1l, s_b1l, s_W1r, s_W2l, s_b2l, s_W2r,
           k_W1l, k_b1l, k_W1r, k_W2l, k_b2l, k_W2r):
    xs = (ui_x, s_x, k_x)
    eis = (ui_edge_index, s_edge_index, k_edge_index)
    W1l = (ui_W1l, s_W1l, k_W1l)
    b1l = (ui_b1l, s_b1l, k_b1l)
    W1r = (ui_W1r, s_W1r, k_W1r)
    W2l = (ui_W2l, s_W2l, k_W2l)
    b2l = (ui_b2l, s_b2l, k_b2l)
    W2r = (ui_W2r, s_W2r, k_W2r)

    PADW = EPT_P - EPT
    if PADW:
        pad_dst = (N + (jnp.arange(PADW)[None, :]
                        + 7 * jnp.arange(NW)[:, None]) % (NP - N)
                   ).astype(jnp.int32)
    src2s, dsts = [], []
    for ei in eis:
        src = ei[0].astype(jnp.int32).reshape(NW, EPT)
        src = jnp.pad(src, ((0, 0), (0, PADW)))
        src2s.append(jnp.stack([2 * src, 2 * src + 1],
                               axis=1).reshape(NW, 2, NCH, CH))
        dst = ei[1].astype(jnp.int32).reshape(NW, EPT)
        if PADW:
            dst = jnp.concatenate([dst, pad_dst], axis=1)
        dsts.append(dst.reshape(NW, NCH, CH))
    dst_flat = jnp.stack([ei[1].astype(jnp.int32).reshape(NW, EPT)
                          for ei in eis])

    cnt_part = _degree_counts(dst_flat)
    cnt = cnt_part.sum(axis=1)
    inv = 1.0 / jnp.clip(cnt, 1.0, None)
    invs = [inv[r][:, None] for r in range(3)]

    z1 = _lin3(xs[0], W1l[0], xs[1], W1l[1], xs[2], W1l[2])

    def chained_seg(z, r, tok):
        z, _ = lax.optimization_barrier((z, tok))
        p = _seg_sum(z.reshape(2 * N, DH), src2s[r], dsts[r])
        halves = (p[0].reshape(NC, NP, DH), p[1].reshape(NC, NP, DH))
        return halves, p[0, 0, 0, :8]

    tok = cnt_part[0, 0, :8]
    p1s, hs, z2s, p2s = [], [], [], []
    for r in range(3):
        p1, tok = chained_seg(z1[r], r, tok)
        p1s.append(p1)
    for r in range(3):
        h, z2 = _combine(p1s[r][0], p1s[r][1], invs[r], xs[r], W1r[r],
                         b1l[r].reshape(1, D), W2l[r])
        hs.append(h)
        z2s.append(z2)
    for r in range(3):
        p2, tok = chained_seg(z2s[r], r, tok)
        p2s.append(p2)

    return _final(
        p2s[0][0], p2s[0][1], invs[0], hs[0], W2r[0], b2l[0].reshape(1, D),
        p2s[1][0], p2s[1][1], invs[1], hs[1], W2r[1], b2l[1].reshape(1, D),
        p2s[2][0], p2s[2][1], invs[2], hs[2], W2r[2], b2l[2].reshape(1, D),
    )
